# baseline (speedup 1.0000x reference)
import functools

import jax
import jax.numpy as jnp
from jax import lax
from jax.experimental import pallas as pl
from jax.experimental.pallas import tpu as pltpu
from jax.experimental.pallas import tpu_sc as plsc

N = 10000
E = 320000
NC = 2
NS = 16
NW = NC * NS
EPT = E // NW
ROWS_A = 624
CH = 16
NCHUNK = EPT // CH
ACH = 80
ANCHUNK = EPT // ACH

_SC_MESH = dict(core_axis_name="c", subcore_axis_name="s")


def _worker_id():
    return lax.axis_index("s") * NC + lax.axis_index("c")


def _deg_body(dst_hbm, ew_hbm, out_hbm, dst_v, ew_v, deg_v):
    wid = _worker_id()
    base = wid * EPT
    pltpu.sync_copy(dst_hbm.at[pl.ds(base, EPT)], dst_v)
    pltpu.sync_copy(ew_hbm.at[pl.ds(base, EPT)], ew_v)

    def zero(i, _):
        deg_v[pl.ds(i * 16, 16)] = jnp.zeros((16,), jnp.float32)
        return 0

    lax.fori_loop(0, N // 16, zero, 0)

    def edge(i, _):
        off = i * CH
        dv = dst_v[pl.ds(off, CH)]
        wv = ew_v[pl.ds(off, CH)]
        plsc.addupdate_scatter(deg_v, [dv], wv)
        return 0

    lax.fori_loop(0, NCHUNK, edge, 0)
    pltpu.sync_copy(deg_v, out_hbm.at[pl.ds(wid * N, N)])


def _deg_call(dst, ew):
    fn = pl.kernel(
        _deg_body,
        out_type=jax.ShapeDtypeStruct((NW * N,), jnp.float32),
        mesh=plsc.VectorSubcoreMesh(**_SC_MESH),
        compiler_params=pltpu.CompilerParams(needs_layout_passes=False),
        scratch_types=[
            pltpu.VMEM((EPT,), jnp.int32),
            pltpu.VMEM((EPT,), jnp.float32),
            pltpu.VMEM((N,), jnp.float32),
        ],
    )
    return fn(dst, ew)


def _agg_body(hp_hbm, src_hbm, dst_hbm, ew_hbm, out_hbm,
              src_v, dst_v, ew_v, rows_a, rows_b, acc_sh,
              sem_a, sem_b, sem_sa, sem_sb, *, D):
    c = lax.axis_index("c")
    s = lax.axis_index("s")
    wid = s * NC + c
    base = wid * EPT
    pltpu.sync_copy(src_hbm.at[pl.ds(base, EPT)], src_v)
    pltpu.sync_copy(dst_hbm.at[pl.ds(base, EPT)], dst_v)
    pltpu.sync_copy(ew_hbm.at[pl.ds(base, EPT)], ew_v)

    zrow = jnp.zeros((16,), jnp.float32)
    for i in range(ACH):
        for j in range(D // 16):
            rows_a[i, pl.ds(j * 16, 16)] = zrow
    row0 = s * ROWS_A

    def zloop(i, _):
        pltpu.sync_copy(rows_a, acc_sh.at[pl.ds(row0 + i * ACH, ACH)])
        return 0

    lax.fori_loop(0, ROWS_A // ACH, zloop, 0)
    pltpu.sync_copy(rows_a.at[pl.ds(0, ROWS_A % ACH)],
                    acc_sh.at[pl.ds(row0 + (ROWS_A // ACH) * ACH,
                                    ROWS_A % ACH)])

    @pl.when(s == NS - 1)
    def _():
        pltpu.sync_copy(rows_a.at[pl.ds(0, 16)],
                        acc_sh.at[pl.ds(NS * ROWS_A, 16)])

    plsc.subcore_barrier()

    def issue_gather(c, buf, sem):
        pltpu.async_copy(hp_hbm.at[src_v.at[pl.ds(c * ACH, ACH)]], buf, sem)

    def wait_gather(c, buf, sem):
        pltpu.make_async_copy(
            hp_hbm.at[src_v.at[pl.ds(c * ACH, ACH)]], buf, sem).wait()

    def scale_and_fire(c, buf, ssem):
        off = c * ACH
        for g in range(ACH // 16):
            goff = off + g * 16
            for r in range(16):
                scale = plsc.load_gather(
                    ew_v, [jnp.full((16,), goff, jnp.int32) + r])
                row = g * 16 + r
                for j in range(D // 16):
                    sl = pl.ds(j * 16, 16)
                    buf[row, sl] = buf[row, sl] * scale
            dv = dst_v[pl.ds(goff, 16)]
            pltpu.async_copy(
                buf.at[pl.ds(g * 16, 16)], acc_sh.at[dv], ssem, add=True)

    def drain_scatters(c, buf, ssem):
        off = c * ACH
        for g in range(ACH // 16):
            dv = dst_v[pl.ds(off + g * 16, 16)]
            pltpu.make_async_copy(
                buf.at[pl.ds(g * 16, 16)], acc_sh.at[dv], ssem).wait()

    issue_gather(0, rows_a, sem_a)
    issue_gather(1, rows_b, sem_b)

    def pair(i, _):
        c0 = i * 2
        wait_gather(c0, rows_a, sem_a)
        scale_and_fire(c0, rows_a, sem_sa)
        wait_gather(c0 + 1, rows_b, sem_b)
        scale_and_fire(c0 + 1, rows_b, sem_sb)
        drain_scatters(c0, rows_a, sem_sa)
        issue_gather(c0 + 2, rows_a, sem_a)

        @pl.when(c0 + 3 < ANCHUNK)
        def _():
            drain_scatters(c0 + 1, rows_b, sem_sb)
            issue_gather(c0 + 3, rows_b, sem_b)

        return 0

    lax.fori_loop(0, (ANCHUNK - 1) // 2, pair, 0)
    drain_scatters(ANCHUNK - 2, rows_b, sem_sb)
    wait_gather(ANCHUNK - 1, rows_a, sem_a)
    scale_and_fire(ANCHUNK - 1, rows_a, sem_sa)
    drain_scatters(ANCHUNK - 1, rows_a, sem_sa)
    plsc.subcore_barrier()
    pltpu.sync_copy(acc_sh.at[pl.ds(row0, ROWS_A)],
                    out_hbm.at[c, pl.ds(row0, ROWS_A)])

    @pl.when(s == NS - 1)
    def _():
        pltpu.sync_copy(acc_sh.at[pl.ds(NS * ROWS_A, 16)],
                        out_hbm.at[c, pl.ds(NS * ROWS_A, 16)])


def _agg_call(hp, src, dst, ew, D):
    fn = pl.kernel(
        functools.partial(_agg_body, D=D),
        out_type=jax.ShapeDtypeStruct((NC, N, D), jnp.float32),
        mesh=plsc.VectorSubcoreMesh(**_SC_MESH),
        compiler_params=pltpu.CompilerParams(needs_layout_passes=False),
        scratch_types=[
            pltpu.VMEM((EPT,), jnp.int32),
            pltpu.VMEM((EPT,), jnp.int32),
            pltpu.VMEM((EPT,), jnp.float32),
            pltpu.VMEM((ACH, D), jnp.float32),
            pltpu.VMEM((ACH, D), jnp.float32),
            pltpu.VMEM_SHARED((N, D), jnp.float32),
            pltpu.SemaphoreType.DMA,
            pltpu.SemaphoreType.DMA,
            pltpu.SemaphoreType.DMA,
            pltpu.SemaphoreType.DMA,
        ],
    )
    return fn(hp, src, dst, ew)


_BR = 512
_GRID = (N + _BR - 1) // _BR


def _mm1_body(degp_ref, x_ref, w_ref, hp_ref, dinv_ref):
    deg = jnp.sum(degp_ref[...], axis=0) + 1.0
    dinv = jnp.where(deg > 0, lax.rsqrt(deg), 0.0)
    h = jnp.dot(x_ref[...], w_ref[...], preferred_element_type=jnp.float32)
    hp_ref[...] = h * dinv[:, None]
    dinv_ref[...] = dinv[:, None]


def _mm1_call(degp, x, W1):
    d_in = x.shape[1]
    d_hid = W1.shape[1]
    return pl.pallas_call(
        _mm1_body,
        grid=(_GRID,),
        in_specs=[
            pl.BlockSpec((NW, _BR), lambda i: (0, i)),
            pl.BlockSpec((_BR, d_in), lambda i: (i, 0)),
            pl.BlockSpec((d_in, d_hid), lambda i: (0, 0)),
        ],
        out_specs=[
            pl.BlockSpec((_BR, d_hid), lambda i: (i, 0)),
            pl.BlockSpec((_BR, 1), lambda i: (i, 0)),
        ],
        out_shape=[
            jax.ShapeDtypeStruct((N, d_hid), jnp.float32),
            jax.ShapeDtypeStruct((N, 1), jnp.float32),
        ],
    )(degp, x, W1)


def _mid_body(acc_ref, hp1_ref, dinv_ref, b1_ref, out_ref):
    a = acc_ref[0] + acc_ref[1] + hp1_ref[...]
    h = jnp.maximum(a * dinv_ref[...] + b1_ref[...], 0.0)
    out_ref[...] = h * dinv_ref[...]


def _mid_call(acc1, hp1, dinv, b1):
    d_hid = hp1.shape[1]
    return pl.pallas_call(
        _mid_body,
        grid=(_GRID,),
        in_specs=[
            pl.BlockSpec((NC, _BR, d_hid), lambda i: (0, i, 0)),
            pl.BlockSpec((_BR, d_hid), lambda i: (i, 0)),
            pl.BlockSpec((_BR, 1), lambda i: (i, 0)),
            pl.BlockSpec((1, d_hid), lambda i: (0, 0)),
        ],
        out_specs=pl.BlockSpec((_BR, d_hid), lambda i: (i, 0)),
        out_shape=jax.ShapeDtypeStruct((N, d_hid), jnp.float32),
    )(acc1, hp1, dinv, b1)


def _fin_body(acc_ref, g2_ref, dinv_ref, w2_ref, b2_ref, out_ref):
    z = (acc_ref[0] + acc_ref[1] + g2_ref[...]) * dinv_ref[...]
    a = jnp.dot(z, w2_ref[...], preferred_element_type=jnp.float32) + b2_ref[...]
    m = jnp.max(a, axis=1, keepdims=True)
    ex = jnp.exp(a - m)
    lse = jnp.log(jnp.sum(ex, axis=1, keepdims=True))
    out_ref[...] = a - m - lse


def _fin_call(acc2, g2, dinv, W2, b2):
    d_hid = g2.shape[1]
    d_out = W2.shape[1]
    return pl.pallas_call(
        _fin_body,
        grid=(_GRID,),
        in_specs=[
            pl.BlockSpec((NC, _BR, d_hid), lambda i: (0, i, 0)),
            pl.BlockSpec((_BR, d_hid), lambda i: (i, 0)),
            pl.BlockSpec((_BR, 1), lambda i: (i, 0)),
            pl.BlockSpec((d_hid, d_out), lambda i: (0, 0)),
            pl.BlockSpec((1, d_out), lambda i: (0, 0)),
        ],
        out_specs=pl.BlockSpec((_BR, d_out), lambda i: (i, 0)),
        out_shape=jax.ShapeDtypeStruct((N, d_out), jnp.float32),
    )(acc2, g2, dinv, W2, b2)


def kernel(x, edge_index, edge_attr, W1, b1, W2, b2):
    src = edge_index[0].astype(jnp.int32)
    dst = edge_index[1].astype(jnp.int32)
    ew = edge_attr.astype(jnp.float32)

    degp = _deg_call(dst, ew).reshape(NW, N)
    hp1, dinv = _mm1_call(degp, x, W1)
    acc1 = _agg_call(hp1, src, dst, ew, W1.shape[1])
    g2 = _mid_call(acc1, hp1, dinv, b1.reshape(1, -1))
    acc2 = _agg_call(g2, src, dst, ew, W1.shape[1])
    return _fin_call(acc2, g2, dinv, W2, b2.reshape(1, -1))


# speedup vs baseline: 1.3161x; 1.3161x over previous
import functools

import jax
import jax.numpy as jnp
from jax import lax
from jax.experimental import pallas as pl
from jax.experimental.pallas import tpu as pltpu
from jax.experimental.pallas import tpu_sc as plsc

N = 10000
E = 320000
NC = 2
NS = 16
NW = NC * NS
EPT = E // NW
ROWS_A = 624
CH = 16
NCHUNK = EPT // CH
ACH = 80
ANCHUNK = EPT // ACH

_SC_MESH = dict(core_axis_name="c", subcore_axis_name="s")


def _worker_id():
    return lax.axis_index("s") * NC + lax.axis_index("c")


def _deg_body(dst_hbm, ew_hbm, out_hbm, dst_v, ew_v, deg_v):
    wid = _worker_id()
    base = wid * EPT
    pltpu.sync_copy(dst_hbm.at[pl.ds(base, EPT)], dst_v)
    pltpu.sync_copy(ew_hbm.at[pl.ds(base, EPT)], ew_v)

    def zero(i, _):
        deg_v[pl.ds(i * 16, 16)] = jnp.zeros((16,), jnp.float32)
        return 0

    lax.fori_loop(0, N // 16, zero, 0)

    def edge(i, _):
        off = i * CH
        dv = dst_v[pl.ds(off, CH)]
        wv = ew_v[pl.ds(off, CH)]
        plsc.addupdate_scatter(deg_v, [dv], wv)
        return 0

    lax.fori_loop(0, NCHUNK, edge, 0)
    pltpu.sync_copy(deg_v, out_hbm.at[pl.ds(wid * N, N)])


def _deg_call(dst, ew):
    fn = pl.kernel(
        _deg_body,
        out_type=jax.ShapeDtypeStruct((NW * N,), jnp.float32),
        mesh=plsc.VectorSubcoreMesh(**_SC_MESH),
        compiler_params=pltpu.CompilerParams(needs_layout_passes=False),
        scratch_types=[
            pltpu.VMEM((EPT,), jnp.int32),
            pltpu.VMEM((EPT,), jnp.float32),
            pltpu.VMEM((N,), jnp.float32),
        ],
    )
    return fn(dst, ew)


def _agg_body(hp_hbm, src_hbm, dst_hbm, ew_hbm, out_hbm,
              src_v, dst_v, ew_v, rows_a, rows_b, acc_sh,
              sem_a, sem_b, sem_sa, sem_sb, *, D):
    c = lax.axis_index("c")
    s = lax.axis_index("s")
    wid = s * NC + c
    base = wid * EPT
    pltpu.sync_copy(src_hbm.at[pl.ds(base, EPT)], src_v)
    pltpu.sync_copy(dst_hbm.at[pl.ds(base, EPT)], dst_v)
    pltpu.sync_copy(ew_hbm.at[pl.ds(base, EPT)], ew_v)

    zrow = jnp.zeros((16,), jnp.float32)
    for i in range(ACH):
        for j in range(D // 16):
            rows_a[i, pl.ds(j * 16, 16)] = zrow
    row0 = s * ROWS_A

    def zloop(i, _):
        pltpu.sync_copy(rows_a, acc_sh.at[pl.ds(row0 + i * ACH, ACH)])
        return 0

    lax.fori_loop(0, ROWS_A // ACH, zloop, 0)
    pltpu.sync_copy(rows_a.at[pl.ds(0, ROWS_A % ACH)],
                    acc_sh.at[pl.ds(row0 + (ROWS_A // ACH) * ACH,
                                    ROWS_A % ACH)])

    @pl.when(s == NS - 1)
    def _():
        pltpu.sync_copy(rows_a.at[pl.ds(0, 16)],
                        acc_sh.at[pl.ds(NS * ROWS_A, 16)])

    plsc.subcore_barrier()

    def issue_gather(c, buf, sem):
        pltpu.async_copy(hp_hbm.at[src_v.at[pl.ds(c * ACH, ACH)]], buf, sem)

    def wait_gather(c, buf, sem):
        pltpu.make_async_copy(
            hp_hbm.at[src_v.at[pl.ds(c * ACH, ACH)]], buf, sem).wait()

    def scale_and_fire(c, buf, ssem):
        off = c * ACH
        for g in range(ACH // 16):
            goff = off + g * 16
            for r in range(16):
                scale = plsc.load_gather(
                    ew_v, [jnp.full((16,), goff, jnp.int32) + r])
                row = g * 16 + r
                for j in range(D // 16):
                    sl = pl.ds(j * 16, 16)
                    buf[row, sl] = buf[row, sl] * scale
            dv = dst_v[pl.ds(goff, 16)]
            pltpu.async_copy(
                buf.at[pl.ds(g * 16, 16)], acc_sh.at[dv], ssem, add=True)

    def drain_scatters(c, buf, ssem):
        off = c * ACH
        for g in range(ACH // 16):
            dv = dst_v[pl.ds(off + g * 16, 16)]
            pltpu.make_async_copy(
                buf.at[pl.ds(g * 16, 16)], acc_sh.at[dv], ssem).wait()

    issue_gather(0, rows_a, sem_a)

    def pair(i, _):
        c0 = i * 2
        issue_gather(c0 + 1, rows_b, sem_b)
        wait_gather(c0, rows_a, sem_a)
        scale_and_fire(c0, rows_a, sem_sa)
        drain_scatters(c0, rows_a, sem_sa)
        issue_gather(c0 + 2, rows_a, sem_a)
        wait_gather(c0 + 1, rows_b, sem_b)
        scale_and_fire(c0 + 1, rows_b, sem_sb)
        drain_scatters(c0 + 1, rows_b, sem_sb)
        return 0

    lax.fori_loop(0, (ANCHUNK - 1) // 2, pair, 0)
    wait_gather(ANCHUNK - 1, rows_a, sem_a)
    scale_and_fire(ANCHUNK - 1, rows_a, sem_sa)
    drain_scatters(ANCHUNK - 1, rows_a, sem_sa)
    plsc.subcore_barrier()
    pltpu.sync_copy(acc_sh.at[pl.ds(row0, ROWS_A)],
                    out_hbm.at[c, pl.ds(row0, ROWS_A)])

    @pl.when(s == NS - 1)
    def _():
        pltpu.sync_copy(acc_sh.at[pl.ds(NS * ROWS_A, 16)],
                        out_hbm.at[c, pl.ds(NS * ROWS_A, 16)])


def _agg_call(hp, src, dst, ew, D):
    fn = pl.kernel(
        functools.partial(_agg_body, D=D),
        out_type=jax.ShapeDtypeStruct((NC, N, D), jnp.float32),
        mesh=plsc.VectorSubcoreMesh(**_SC_MESH),
        compiler_params=pltpu.CompilerParams(needs_layout_passes=False),
        scratch_types=[
            pltpu.VMEM((EPT,), jnp.int32),
            pltpu.VMEM((EPT,), jnp.int32),
            pltpu.VMEM((EPT,), jnp.float32),
            pltpu.VMEM((ACH, D), jnp.float32),
            pltpu.VMEM((ACH, D), jnp.float32),
            pltpu.VMEM_SHARED((N, D), jnp.float32),
            pltpu.SemaphoreType.DMA,
            pltpu.SemaphoreType.DMA,
            pltpu.SemaphoreType.DMA,
            pltpu.SemaphoreType.DMA,
        ],
    )
    return fn(hp, src, dst, ew)


_BR = 512
_GRID = (N + _BR - 1) // _BR


def _mm1_body(degp_ref, x_ref, w_ref, hp_ref, dinv_ref):
    deg = jnp.sum(degp_ref[...], axis=0) + 1.0
    dinv = jnp.where(deg > 0, lax.rsqrt(deg), 0.0)
    h = jnp.dot(x_ref[...], w_ref[...], preferred_element_type=jnp.float32)
    hp_ref[...] = h * dinv[:, None]
    dinv_ref[...] = dinv[:, None]


def _mm1_call(degp, x, W1):
    d_in = x.shape[1]
    d_hid = W1.shape[1]
    return pl.pallas_call(
        _mm1_body,
        grid=(_GRID,),
        in_specs=[
            pl.BlockSpec((NW, _BR), lambda i: (0, i)),
            pl.BlockSpec((_BR, d_in), lambda i: (i, 0)),
            pl.BlockSpec((d_in, d_hid), lambda i: (0, 0)),
        ],
        out_specs=[
            pl.BlockSpec((_BR, d_hid), lambda i: (i, 0)),
            pl.BlockSpec((_BR, 1), lambda i: (i, 0)),
        ],
        out_shape=[
            jax.ShapeDtypeStruct((N, d_hid), jnp.float32),
            jax.ShapeDtypeStruct((N, 1), jnp.float32),
        ],
    )(degp, x, W1)


def _mid_body(acc_ref, hp1_ref, dinv_ref, b1_ref, out_ref):
    a = acc_ref[0] + acc_ref[1] + hp1_ref[...]
    h = jnp.maximum(a * dinv_ref[...] + b1_ref[...], 0.0)
    out_ref[...] = h * dinv_ref[...]


def _mid_call(acc1, hp1, dinv, b1):
    d_hid = hp1.shape[1]
    return pl.pallas_call(
        _mid_body,
        grid=(_GRID,),
        in_specs=[
            pl.BlockSpec((NC, _BR, d_hid), lambda i: (0, i, 0)),
            pl.BlockSpec((_BR, d_hid), lambda i: (i, 0)),
            pl.BlockSpec((_BR, 1), lambda i: (i, 0)),
            pl.BlockSpec((1, d_hid), lambda i: (0, 0)),
        ],
        out_specs=pl.BlockSpec((_BR, d_hid), lambda i: (i, 0)),
        out_shape=jax.ShapeDtypeStruct((N, d_hid), jnp.float32),
    )(acc1, hp1, dinv, b1)


def _fin_body(acc_ref, g2_ref, dinv_ref, w2_ref, b2_ref, out_ref):
    z = (acc_ref[0] + acc_ref[1] + g2_ref[...]) * dinv_ref[...]
    a = jnp.dot(z, w2_ref[...], preferred_element_type=jnp.float32) + b2_ref[...]
    m = jnp.max(a, axis=1, keepdims=True)
    ex = jnp.exp(a - m)
    lse = jnp.log(jnp.sum(ex, axis=1, keepdims=True))
    out_ref[...] = a - m - lse


def _fin_call(acc2, g2, dinv, W2, b2):
    d_hid = g2.shape[1]
    d_out = W2.shape[1]
    return pl.pallas_call(
        _fin_body,
        grid=(_GRID,),
        in_specs=[
            pl.BlockSpec((NC, _BR, d_hid), lambda i: (0, i, 0)),
            pl.BlockSpec((_BR, d_hid), lambda i: (i, 0)),
            pl.BlockSpec((_BR, 1), lambda i: (i, 0)),
            pl.BlockSpec((d_hid, d_out), lambda i: (0, 0)),
            pl.BlockSpec((1, d_out), lambda i: (0, 0)),
        ],
        out_specs=pl.BlockSpec((_BR, d_out), lambda i: (i, 0)),
        out_shape=jax.ShapeDtypeStruct((N, d_out), jnp.float32),
    )(acc2, g2, dinv, W2, b2)


def kernel(x, edge_index, edge_attr, W1, b1, W2, b2):
    src = edge_index[0].astype(jnp.int32)
    dst = edge_index[1].astype(jnp.int32)
    ew = edge_attr.astype(jnp.float32)

    degp = _deg_call(dst, ew).reshape(NW, N)
    hp1, dinv = _mm1_call(degp, x, W1)
    acc1 = _agg_call(hp1, src, dst, ew, W1.shape[1])
    g2 = _mid_call(acc1, hp1, dinv, b1.reshape(1, -1))
    acc2 = _agg_call(g2, src, dst, ew, W1.shape[1])
    return _fin_call(acc2, g2, dinv, W2, b2.reshape(1, -1))


# speedup vs baseline: 2.1588x; 1.6403x over previous
import functools

import jax
import jax.numpy as jnp
from jax import lax
from jax.experimental import pallas as pl
from jax.experimental.pallas import tpu as pltpu
from jax.experimental.pallas import tpu_sc as plsc

N = 10000
E = 320000
NC = 2
NS = 16
NW = NC * NS
EPT = E // NW
ROWS_A = 624
CH = 16
NCHUNK = EPT // CH
ACH = 80
ANCHUNK = EPT // ACH

_SC_MESH = dict(core_axis_name="c", subcore_axis_name="s")


def _worker_id():
    return lax.axis_index("s") * NC + lax.axis_index("c")


def _deg_body(dst_hbm, ew_hbm, out_hbm, dst_v, ew_v, deg_v):
    wid = _worker_id()
    base = wid * EPT
    pltpu.sync_copy(dst_hbm.at[pl.ds(base, EPT)], dst_v)
    pltpu.sync_copy(ew_hbm.at[pl.ds(base, EPT)], ew_v)

    def zero(i, _):
        deg_v[pl.ds(i * 16, 16)] = jnp.zeros((16,), jnp.float32)
        return 0

    lax.fori_loop(0, N // 16, zero, 0)

    def edge(i, _):
        off = i * CH
        dv = dst_v[pl.ds(off, CH)]
        wv = ew_v[pl.ds(off, CH)]
        plsc.addupdate_scatter(deg_v, [dv], wv)
        return 0

    lax.fori_loop(0, NCHUNK, edge, 0)
    pltpu.sync_copy(deg_v, out_hbm.at[pl.ds(wid * N, N)])


def _deg_call(dst, ew):
    fn = pl.kernel(
        _deg_body,
        out_type=jax.ShapeDtypeStruct((NW * N,), jnp.float32),
        mesh=plsc.VectorSubcoreMesh(**_SC_MESH),
        compiler_params=pltpu.CompilerParams(needs_layout_passes=False),
        scratch_types=[
            pltpu.VMEM((EPT,), jnp.int32),
            pltpu.VMEM((EPT,), jnp.float32),
            pltpu.VMEM((N,), jnp.float32),
        ],
    )
    return fn(dst, ew)


def _agg_body(hp_hbm, src_hbm, dst_hbm, ew_hbm, out_hbm,
              src_v, dst_v, ew_v, rows_a, rows_b, acc_sh,
              sem_a, sem_b, sem_sa, sem_sb, *, D):
    c = lax.axis_index("c")
    s = lax.axis_index("s")
    wid = s * NC + c
    base = wid * EPT
    pltpu.sync_copy(src_hbm.at[pl.ds(base, EPT)], src_v)
    pltpu.sync_copy(dst_hbm.at[pl.ds(base, EPT)], dst_v)
    pltpu.sync_copy(ew_hbm.at[pl.ds(base, EPT)], ew_v)

    zrow = jnp.zeros((16,), jnp.float32)
    for i in range(ACH):
        for j in range(D // 16):
            rows_a[i, pl.ds(j * 16, 16)] = zrow
    row0 = s * ROWS_A

    def zloop(i, _):
        pltpu.sync_copy(rows_a, acc_sh.at[pl.ds(row0 + i * ACH, ACH)])
        return 0

    lax.fori_loop(0, ROWS_A // ACH, zloop, 0)
    pltpu.sync_copy(rows_a.at[pl.ds(0, ROWS_A % ACH)],
                    acc_sh.at[pl.ds(row0 + (ROWS_A // ACH) * ACH,
                                    ROWS_A % ACH)])

    @pl.when(s == NS - 1)
    def _():
        pltpu.sync_copy(rows_a.at[pl.ds(0, 16)],
                        acc_sh.at[pl.ds(NS * ROWS_A, 16)])

    plsc.subcore_barrier()

    def issue_gather(c, buf, sem):
        pltpu.async_copy(hp_hbm.at[src_v.at[pl.ds(c * ACH, ACH)]], buf, sem)

    def wait_gather(c, buf, sem):
        pltpu.make_async_copy(
            hp_hbm.at[src_v.at[pl.ds(c * ACH, ACH)]], buf, sem).wait()

    def scale_and_fire(c, buf, ssem):
        off = c * ACH
        for g in range(ACH // 16):
            goff = off + g * 16
            ewv = ew_v[pl.ds(goff, 16)]
            for r in range(16):
                scale = jnp.take_along_axis(
                    ewv, jnp.full((16,), r, jnp.int32), axis=0, mode="fill")
                row = g * 16 + r
                for j in range(D // 16):
                    sl = pl.ds(j * 16, 16)
                    buf[row, sl] = buf[row, sl] * scale
            dv = dst_v[pl.ds(goff, 16)]
            pltpu.async_copy(
                buf.at[pl.ds(g * 16, 16)], acc_sh.at[dv], ssem, add=True)

    def drain_scatters(c, buf, ssem):
        off = c * ACH
        for g in range(ACH // 16):
            dv = dst_v[pl.ds(off + g * 16, 16)]
            pltpu.make_async_copy(
                buf.at[pl.ds(g * 16, 16)], acc_sh.at[dv], ssem).wait()

    issue_gather(0, rows_a, sem_a)

    def pair(i, _):
        c0 = i * 2
        issue_gather(c0 + 1, rows_b, sem_b)
        wait_gather(c0, rows_a, sem_a)
        scale_and_fire(c0, rows_a, sem_sa)
        drain_scatters(c0, rows_a, sem_sa)
        issue_gather(c0 + 2, rows_a, sem_a)
        wait_gather(c0 + 1, rows_b, sem_b)
        scale_and_fire(c0 + 1, rows_b, sem_sb)
        drain_scatters(c0 + 1, rows_b, sem_sb)
        return 0

    lax.fori_loop(0, (ANCHUNK - 1) // 2, pair, 0)
    wait_gather(ANCHUNK - 1, rows_a, sem_a)
    scale_and_fire(ANCHUNK - 1, rows_a, sem_sa)
    drain_scatters(ANCHUNK - 1, rows_a, sem_sa)
    plsc.subcore_barrier()
    pltpu.sync_copy(acc_sh.at[pl.ds(row0, ROWS_A)],
                    out_hbm.at[c, pl.ds(row0, ROWS_A)])

    @pl.when(s == NS - 1)
    def _():
        pltpu.sync_copy(acc_sh.at[pl.ds(NS * ROWS_A, 16)],
                        out_hbm.at[c, pl.ds(NS * ROWS_A, 16)])


def _agg_call(hp, src, dst, ew, D):
    fn = pl.kernel(
        functools.partial(_agg_body, D=D),
        out_type=jax.ShapeDtypeStruct((NC, N, D), jnp.float32),
        mesh=plsc.VectorSubcoreMesh(**_SC_MESH),
        compiler_params=pltpu.CompilerParams(needs_layout_passes=False),
        scratch_types=[
            pltpu.VMEM((EPT,), jnp.int32),
            pltpu.VMEM((EPT,), jnp.int32),
            pltpu.VMEM((EPT,), jnp.float32),
            pltpu.VMEM((ACH, D), jnp.float32),
            pltpu.VMEM((ACH, D), jnp.float32),
            pltpu.VMEM_SHARED((N, D), jnp.float32),
            pltpu.SemaphoreType.DMA,
            pltpu.SemaphoreType.DMA,
            pltpu.SemaphoreType.DMA,
            pltpu.SemaphoreType.DMA,
        ],
    )
    return fn(hp, src, dst, ew)


_BR = 512
_GRID = (N + _BR - 1) // _BR


def _mm1_body(degp_ref, x_ref, w_ref, hp_ref, dinv_ref):
    deg = jnp.sum(degp_ref[...], axis=0) + 1.0
    dinv = jnp.where(deg > 0, lax.rsqrt(deg), 0.0)
    h = jnp.dot(x_ref[...], w_ref[...], preferred_element_type=jnp.float32)
    hp_ref[...] = h * dinv[:, None]
    dinv_ref[...] = dinv[:, None]


def _mm1_call(degp, x, W1):
    d_in = x.shape[1]
    d_hid = W1.shape[1]
    return pl.pallas_call(
        _mm1_body,
        grid=(_GRID,),
        in_specs=[
            pl.BlockSpec((NW, _BR), lambda i: (0, i)),
            pl.BlockSpec((_BR, d_in), lambda i: (i, 0)),
            pl.BlockSpec((d_in, d_hid), lambda i: (0, 0)),
        ],
        out_specs=[
            pl.BlockSpec((_BR, d_hid), lambda i: (i, 0)),
            pl.BlockSpec((_BR, 1), lambda i: (i, 0)),
        ],
        out_shape=[
            jax.ShapeDtypeStruct((N, d_hid), jnp.float32),
            jax.ShapeDtypeStruct((N, 1), jnp.float32),
        ],
    )(degp, x, W1)


def _mid_body(acc_ref, hp1_ref, dinv_ref, b1_ref, out_ref):
    a = acc_ref[0] + acc_ref[1] + hp1_ref[...]
    h = jnp.maximum(a * dinv_ref[...] + b1_ref[...], 0.0)
    out_ref[...] = h * dinv_ref[...]


def _mid_call(acc1, hp1, dinv, b1):
    d_hid = hp1.shape[1]
    return pl.pallas_call(
        _mid_body,
        grid=(_GRID,),
        in_specs=[
            pl.BlockSpec((NC, _BR, d_hid), lambda i: (0, i, 0)),
            pl.BlockSpec((_BR, d_hid), lambda i: (i, 0)),
            pl.BlockSpec((_BR, 1), lambda i: (i, 0)),
            pl.BlockSpec((1, d_hid), lambda i: (0, 0)),
        ],
        out_specs=pl.BlockSpec((_BR, d_hid), lambda i: (i, 0)),
        out_shape=jax.ShapeDtypeStruct((N, d_hid), jnp.float32),
    )(acc1, hp1, dinv, b1)


def _fin_body(acc_ref, g2_ref, dinv_ref, w2_ref, b2_ref, out_ref):
    z = (acc_ref[0] + acc_ref[1] + g2_ref[...]) * dinv_ref[...]
    a = jnp.dot(z, w2_ref[...], preferred_element_type=jnp.float32) + b2_ref[...]
    m = jnp.max(a, axis=1, keepdims=True)
    ex = jnp.exp(a - m)
    lse = jnp.log(jnp.sum(ex, axis=1, keepdims=True))
    out_ref[...] = a - m - lse


def _fin_call(acc2, g2, dinv, W2, b2):
    d_hid = g2.shape[1]
    d_out = W2.shape[1]
    return pl.pallas_call(
        _fin_body,
        grid=(_GRID,),
        in_specs=[
            pl.BlockSpec((NC, _BR, d_hid), lambda i: (0, i, 0)),
            pl.BlockSpec((_BR, d_hid), lambda i: (i, 0)),
            pl.BlockSpec((_BR, 1), lambda i: (i, 0)),
            pl.BlockSpec((d_hid, d_out), lambda i: (0, 0)),
            pl.BlockSpec((1, d_out), lambda i: (0, 0)),
        ],
        out_specs=pl.BlockSpec((_BR, d_out), lambda i: (i, 0)),
        out_shape=jax.ShapeDtypeStruct((N, d_out), jnp.float32),
    )(acc2, g2, dinv, W2, b2)


def kernel(x, edge_index, edge_attr, W1, b1, W2, b2):
    src = edge_index[0].astype(jnp.int32)
    dst = edge_index[1].astype(jnp.int32)
    ew = edge_attr.astype(jnp.float32)

    degp = _deg_call(dst, ew).reshape(NW, N)
    hp1, dinv = _mm1_call(degp, x, W1)
    acc1 = _agg_call(hp1, src, dst, ew, W1.shape[1])
    g2 = _mid_call(acc1, hp1, dinv, b1.reshape(1, -1))
    acc2 = _agg_call(g2, src, dst, ew, W1.shape[1])
    return _fin_call(acc2, g2, dinv, W2, b2.reshape(1, -1))
